# baseline (speedup 1.0000x reference)
import functools

import jax
import jax.numpy as jnp
from jax import lax
from jax.experimental import pallas as pl
from jax.experimental.pallas import tpu as pltpu
from jax.experimental.pallas import tpu_sc as plsc

VOCAB = 49408
D = 512
N_CLS = 1000
CTX_LEN = 77
M = 16
HDR = M + 1
G = CTX_LEN - HDR
GP = 64
NPAD = 1024
TOKP = 80
L = 16


def _body(tok_hbm, gidx_hbm, pidx_hbm, table_hbm, ctx_hbm,
          emb_hbm, eof_hbm,
          buf_a, buf_b, gidx_v, tok_v, eof_v, pidx_v,
          gsem_a, gsem_b, wsem_a, wsem_b, psem,
          *, nc, cpw):
    wid = lax.axis_index("s") * nc + lax.axis_index("c")
    base = wid * cpw

    def start_gather(c, buf, sem):
        @pl.when(base + c < N_CLS)
        def _():
            pltpu.async_copy(table_hbm.at[gidx_v.at[c]],
                             buf.at[pl.ds(HDR, GP)], sem)

    def wait_gather(c, buf, sem):
        @pl.when(base + c < N_CLS)
        def _():
            pltpu.make_async_copy(table_hbm.at[gidx_v.at[c]],
                                  buf.at[pl.ds(HDR, GP)], sem).wait()

    def start_write(c, buf, sem):
        @pl.when(base + c < N_CLS)
        def _():
            pltpu.async_copy(buf.at[pl.ds(0, CTX_LEN)],
                             emb_hbm.at[base + c], sem)

    def wait_write(c, buf, sem):
        @pl.when(base + c < N_CLS)
        def _():
            pltpu.make_async_copy(buf.at[pl.ds(0, CTX_LEN)],
                                  emb_hbm.at[base + c], sem).wait()

    pltpu.sync_copy(pidx_hbm.at[pl.ds(0, 1)], pidx_v)
    for buf in (buf_a, buf_b):
        pltpu.sync_copy(ctx_hbm, buf.at[pl.ds(1, M)])
        pltpu.async_copy(table_hbm.at[pidx_v], buf.at[pl.ds(0, 1)], psem).wait()

    pltpu.sync_copy(gidx_hbm.at[pl.ds(base, cpw)], gidx_v)
    pltpu.sync_copy(tok_hbm.at[wid], tok_v)

    start_gather(0, buf_a, gsem_a)

    for g in range(cpw // L):
        def eof_body(j, mb, g=g):
            m, best = mb
            v = tok_v[j, pl.ds(g * L, L)]
            gt = v > m
            best = jnp.where(gt, jnp.full((L,), j, jnp.int32), best)
            m = jnp.maximum(m, v)
            return m, best

        m0 = jnp.full((L,), -1, jnp.int32)
        b0 = jnp.zeros((L,), jnp.int32)
        _, best = lax.fori_loop(0, CTX_LEN, eof_body, (m0, b0))
        eof_v[pl.ds(g * L, L)] = best

    pltpu.sync_copy(eof_v, eof_hbm.at[pl.ds(base, cpw)])

    def pair_body(p, carry):
        c0 = 2 * p
        c1 = c0 + 1

        @pl.when(p > 0)
        def _():
            wait_write(c0 - 1, buf_b, wsem_b)
        start_gather(c1, buf_b, gsem_b)
        wait_gather(c0, buf_a, gsem_a)
        start_write(c0, buf_a, wsem_a)
        wait_gather(c1, buf_b, gsem_b)
        start_write(c1, buf_b, wsem_b)

        @pl.when(p < cpw // 2 - 1)
        def _():
            wait_write(c0, buf_a, wsem_a)
            start_gather(c0 + 2, buf_a, gsem_a)
        return carry

    lax.fori_loop(0, cpw // 2, pair_body, 0)
    wait_write(cpw - 2, buf_a, wsem_a)
    wait_write(cpw - 1, buf_b, wsem_b)


def kernel(tokenized_text_prototypes, token_embedding_table, ctx_vectors):
    tokens = tokenized_text_prototypes.astype(jnp.int32)
    gidx = jnp.pad(tokens[:, HDR:], ((0, NPAD - N_CLS), (0, GP - G)))
    tokp = jnp.pad(tokens, ((0, NPAD - N_CLS), (0, TOKP - CTX_LEN)),
                   constant_values=-1)
    pidx = jnp.full((8,), tokens[0, 0], jnp.int32)

    info = plsc.get_sparse_core_info()
    nc, ns = info.num_cores, info.num_subcores
    nw = nc * ns
    cpw = NPAD // nw
    tokt = tokp.reshape(nw, cpw, TOKP).transpose(0, 2, 1)

    mesh = plsc.VectorSubcoreMesh(core_axis_name="c", subcore_axis_name="s",
                                  num_cores=nc, num_subcores=ns)
    fn = pl.kernel(
        functools.partial(_body, nc=nc, cpw=cpw),
        out_type=(
            jax.ShapeDtypeStruct((N_CLS, CTX_LEN, D), jnp.float32),
            jax.ShapeDtypeStruct((NPAD,), jnp.int32),
        ),
        mesh=mesh,
        scratch_types=[
            pltpu.VMEM((HDR + GP, D), jnp.float32),
            pltpu.VMEM((HDR + GP, D), jnp.float32),
            pltpu.VMEM((cpw, GP), jnp.int32),
            pltpu.VMEM((TOKP, cpw), jnp.int32),
            pltpu.VMEM((cpw,), jnp.int32),
            pltpu.VMEM((1,), jnp.int32),
            pltpu.SemaphoreType.DMA,
            pltpu.SemaphoreType.DMA,
            pltpu.SemaphoreType.DMA,
            pltpu.SemaphoreType.DMA,
            pltpu.SemaphoreType.DMA,
        ],
        compiler_params=pltpu.CompilerParams(use_tc_tiling_on_sc=False),
    )
    emb, eof = fn(tokt, gidx, pidx, token_embedding_table, ctx_vectors)
    return emb, eof[:N_CLS]


# speedup vs baseline: 1.9420x; 1.9420x over previous
import functools

import jax
import jax.numpy as jnp
from jax import lax
from jax.experimental import pallas as pl
from jax.experimental.pallas import tpu as pltpu
from jax.experimental.pallas import tpu_sc as plsc

VOCAB = 49408
D = 512
N_CLS = 1000
CTX_LEN = 77
M = 16
HDR = M + 1
TOKP = 80
L = 16


def _body(tjidx_hbm, tokt_hbm, table_hbm, ctx_hbm,
          emb_hbm, eof_hbm,
          buf_a, buf_b, tjidx_v, tok_v, eof_v,
          gsem_a, gsem_b, wsem_a, wsem_b,
          *, nc, cpw):
    wid = lax.axis_index("s") * nc + lax.axis_index("c")
    base = jnp.minimum(wid * cpw, N_CLS - cpw)

    pltpu.sync_copy(tjidx_hbm.at[wid], tjidx_v)
    pltpu.sync_copy(tokt_hbm.at[wid], tok_v)

    def start_gather(j, src, buf, sem):
        pltpu.async_copy(src.at[tjidx_v.at[j]], buf, sem)

    def wait_gather(j, src, buf, sem):
        pltpu.make_async_copy(src.at[tjidx_v.at[j]], buf, sem).wait()

    def start_write(j, buf, sem):
        pltpu.async_copy(buf, emb_hbm.at[j, pl.ds(base, cpw)], sem)

    def wait_write(j, buf, sem):
        pltpu.make_async_copy(buf, emb_hbm.at[j, pl.ds(base, cpw)], sem).wait()

    def run_segment(lo, n, src):
        start_gather(lo, src, buf_a, gsem_a)

        def pair_body(p, carry):
            j0 = lo + 2 * p
            j1 = j0 + 1

            @pl.when(p > 0)
            def _():
                wait_write(j0 - 1, buf_b, wsem_b)
            start_gather(j1, src, buf_b, gsem_b)
            wait_gather(j0, src, buf_a, gsem_a)
            start_write(j0, buf_a, wsem_a)
            wait_gather(j1, src, buf_b, gsem_b)
            start_write(j1, buf_b, wsem_b)

            @pl.when(p < n // 2 - 1)
            def _():
                wait_write(j0, buf_a, wsem_a)
                start_gather(j0 + 2, src, buf_a, gsem_a)
            return carry

        lax.fori_loop(0, n // 2, pair_body, 0)
        wait_write(lo + n - 2, buf_a, wsem_a)
        wait_write(lo + n - 1, buf_b, wsem_b)

    start_gather(0, table_hbm, buf_a, gsem_a)

    for g in range(cpw // L):
        def eof_body(j, mb, g=g):
            m, best = mb
            v = tok_v[j, pl.ds(g * L, L)]
            gt = v > m
            best = jnp.where(gt, jnp.full((L,), j, jnp.int32), best)
            m = jnp.maximum(m, v)
            return m, best

        m0 = jnp.full((L,), -1, jnp.int32)
        b0 = jnp.zeros((L,), jnp.int32)
        _, best = lax.fori_loop(0, CTX_LEN, eof_body, (m0, b0))
        eof_v[pl.ds(g * L, L)] = best

    pltpu.sync_copy(eof_v, eof_hbm.at[pl.ds(base, cpw)])

    wait_gather(0, table_hbm, buf_a, gsem_a)
    start_write(0, buf_a, wsem_a)
    wait_write(0, buf_a, wsem_a)
    run_segment(1, M, ctx_hbm)
    run_segment(HDR, CTX_LEN - HDR, table_hbm)


def kernel(tokenized_text_prototypes, token_embedding_table, ctx_vectors):
    tokens = tokenized_text_prototypes.astype(jnp.int32)

    info = plsc.get_sparse_core_info()
    nc, ns = info.num_cores, info.num_subcores
    nw = nc * ns
    cpw = ((N_CLS + nw - 1) // nw + 7) // 8 * 8

    base = jnp.minimum(jnp.arange(nw) * cpw, N_CLS - cpw)
    cls = base[:, None] + jnp.arange(cpw)[None, :]

    blk = tokens[cls]
    pref = jnp.broadcast_to(tokens[0, 0], (nw, 1, cpw))
    ctxi = jnp.broadcast_to(jnp.arange(M, dtype=jnp.int32)[None, :, None],
                            (nw, M, cpw))
    sufi = blk[:, :, HDR:].transpose(0, 2, 1)
    tjidx = jnp.concatenate([pref, ctxi, sufi], axis=1)

    tokt = jnp.pad(blk, ((0, 0), (0, 0), (0, TOKP - CTX_LEN)),
                   constant_values=-1).transpose(0, 2, 1)

    mesh = plsc.VectorSubcoreMesh(core_axis_name="c", subcore_axis_name="s",
                                  num_cores=nc, num_subcores=ns)
    fn = pl.kernel(
        functools.partial(_body, nc=nc, cpw=cpw),
        out_type=(
            jax.ShapeDtypeStruct((CTX_LEN, N_CLS, D), jnp.float32),
            jax.ShapeDtypeStruct((N_CLS,), jnp.int32),
        ),
        mesh=mesh,
        scratch_types=[
            pltpu.VMEM((cpw, D), jnp.float32),
            pltpu.VMEM((cpw, D), jnp.float32),
            pltpu.VMEM((CTX_LEN, cpw), jnp.int32),
            pltpu.VMEM((TOKP, cpw), jnp.int32),
            pltpu.VMEM((cpw,), jnp.int32),
            pltpu.SemaphoreType.DMA,
            pltpu.SemaphoreType.DMA,
            pltpu.SemaphoreType.DMA,
            pltpu.SemaphoreType.DMA,
        ],
        compiler_params=pltpu.CompilerParams(use_tc_tiling_on_sc=False),
    )
    emb77, eofs = fn(tjidx, tokt, token_embedding_table, ctx_vectors)
    return jnp.transpose(emb77, (1, 0, 2)), eofs


# speedup vs baseline: 2.9613x; 1.5249x over previous
import functools

import jax
import jax.numpy as jnp
from jax import lax
from jax.experimental import pallas as pl
from jax.experimental.pallas import tpu as pltpu
from jax.experimental.pallas import tpu_sc as plsc

VOCAB = 49408
D = 512
N_CLS = 1000
CTX_LEN = 77
M = 16
HDR = M + 1
TOKP = 80
L = 16


def _body(tjidx_hbm, tokt_hbm, table_hbm, ctx_hbm,
          emb_hbm, eof_hbm,
          buf_a, buf_b, tjidx_v, tok_v, eof_v,
          gsem_a, gsem_b, wsem_a, wsem_b,
          *, nc, cpw):
    wid = lax.axis_index("s") * nc + lax.axis_index("c")
    base = jnp.minimum(wid * cpw, N_CLS - cpw)

    pltpu.sync_copy(tjidx_hbm.at[pl.ds(wid * (CTX_LEN * cpw), CTX_LEN * cpw)],
                    tjidx_v)
    pltpu.sync_copy(tokt_hbm.at[pl.ds(wid * (TOKP * cpw), TOKP * cpw)], tok_v)

    def start_gather(j, src, buf, sem):
        pltpu.async_copy(src.at[tjidx_v.at[pl.ds(j * cpw, cpw)]], buf, sem)

    def wait_gather(j, src, buf, sem):
        pltpu.make_async_copy(src.at[tjidx_v.at[pl.ds(j * cpw, cpw)]],
                              buf, sem).wait()

    def start_write(j, buf, sem):
        pltpu.async_copy(buf, emb_hbm.at[j, pl.ds(base, cpw)], sem)

    def wait_write(j, buf, sem):
        pltpu.make_async_copy(buf, emb_hbm.at[j, pl.ds(base, cpw)], sem).wait()

    def run_segment(lo, n, src):
        start_gather(lo, src, buf_a, gsem_a)

        def pair_body(p, carry):
            j0 = lo + 2 * p
            j1 = j0 + 1

            @pl.when(p > 0)
            def _():
                wait_write(j0 - 1, buf_b, wsem_b)
            start_gather(j1, src, buf_b, gsem_b)
            wait_gather(j0, src, buf_a, gsem_a)
            start_write(j0, buf_a, wsem_a)
            wait_gather(j1, src, buf_b, gsem_b)
            start_write(j1, buf_b, wsem_b)

            @pl.when(p < n // 2 - 1)
            def _():
                wait_write(j0, buf_a, wsem_a)
                start_gather(j0 + 2, src, buf_a, gsem_a)
            return carry

        lax.fori_loop(0, n // 2, pair_body, 0)
        wait_write(lo + n - 2, buf_a, wsem_a)
        wait_write(lo + n - 1, buf_b, wsem_b)

    start_gather(0, table_hbm, buf_a, gsem_a)

    for g in range(cpw // L):
        def eof_body(j, mb, g=g):
            m, best = mb
            v = tok_v[pl.ds(j * cpw + g * L, L)]
            gt = v > m
            best = jnp.where(gt, jnp.full((L,), j, jnp.int32), best)
            m = jnp.maximum(m, v)
            return m, best

        m0 = jnp.full((L,), -1, jnp.int32)
        b0 = jnp.zeros((L,), jnp.int32)
        _, best = lax.fori_loop(0, CTX_LEN, eof_body, (m0, b0))
        eof_v[pl.ds(g * L, L)] = best

    pltpu.sync_copy(eof_v, eof_hbm.at[pl.ds(base, cpw)])

    wait_gather(0, table_hbm, buf_a, gsem_a)
    start_write(0, buf_a, wsem_a)
    wait_write(0, buf_a, wsem_a)
    run_segment(1, M, ctx_hbm)
    run_segment(HDR, CTX_LEN - HDR, table_hbm)


def kernel(tokenized_text_prototypes, token_embedding_table, ctx_vectors):
    tokens = tokenized_text_prototypes.astype(jnp.int32)

    info = plsc.get_sparse_core_info()
    nc, ns = info.num_cores, info.num_subcores
    nw = nc * ns
    cpw = ((N_CLS + nw - 1) // nw + 7) // 8 * 8

    base = jnp.minimum(jnp.arange(nw) * cpw, N_CLS - cpw)
    cls = base[:, None] + jnp.arange(cpw)[None, :]

    blk = tokens[cls]
    pref = jnp.broadcast_to(tokens[0, 0], (nw, 1, cpw))
    ctxi = jnp.broadcast_to(jnp.arange(M, dtype=jnp.int32)[None, :, None],
                            (nw, M, cpw))
    sufi = blk[:, :, HDR:].transpose(0, 2, 1)
    tjidx = jnp.concatenate([pref, ctxi, sufi], axis=1)

    tokt = jnp.pad(blk, ((0, 0), (0, 0), (0, TOKP - CTX_LEN)),
                   constant_values=-1).transpose(0, 2, 1)

    mesh = plsc.VectorSubcoreMesh(core_axis_name="c", subcore_axis_name="s",
                                  num_cores=nc, num_subcores=ns)
    fn = pl.kernel(
        functools.partial(_body, nc=nc, cpw=cpw),
        out_type=(
            jax.ShapeDtypeStruct((CTX_LEN, N_CLS, D), jnp.float32),
            jax.ShapeDtypeStruct((N_CLS,), jnp.int32),
        ),
        mesh=mesh,
        scratch_types=[
            pltpu.VMEM((cpw, D), jnp.float32),
            pltpu.VMEM((cpw, D), jnp.float32),
            pltpu.VMEM((CTX_LEN * cpw,), jnp.int32),
            pltpu.VMEM((TOKP * cpw,), jnp.int32),
            pltpu.VMEM((cpw,), jnp.int32),
            pltpu.SemaphoreType.DMA,
            pltpu.SemaphoreType.DMA,
            pltpu.SemaphoreType.DMA,
            pltpu.SemaphoreType.DMA,
        ],
        compiler_params=pltpu.CompilerParams(use_tc_tiling_on_sc=True),
    )
    emb77, eofs = fn(tjidx.reshape(-1), tokt.reshape(-1),
                     token_embedding_table, ctx_vectors)
    return jnp.transpose(emb77, (1, 0, 2)), eofs


# speedup vs baseline: 3.0453x; 1.0284x over previous
import functools

import jax
import jax.numpy as jnp
from jax import lax
from jax.experimental import pallas as pl
from jax.experimental.pallas import tpu as pltpu
from jax.experimental.pallas import tpu_sc as plsc

VOCAB = 49408
D = 512
N_CLS = 1000
CTX_LEN = 77
M = 16
HDR = M + 1
TOKP = 80
L = 16


NB = 4


def _body(tjidx_hbm, tokt_hbm, table_hbm, ctx_hbm,
          emb_hbm, eof_hbm,
          bufs, tjidx_v, tok_v, eof_v, gsems, wsems,
          *, nc, cpw):
    wid = lax.axis_index("s") * nc + lax.axis_index("c")
    base = jnp.minimum(wid * cpw, N_CLS - cpw)

    pltpu.sync_copy(tjidx_hbm.at[pl.ds(wid * (CTX_LEN * cpw), CTX_LEN * cpw)],
                    tjidx_v)
    pltpu.sync_copy(tokt_hbm.at[pl.ds(wid * (TOKP * cpw), TOKP * cpw)], tok_v)

    def src(j):
        return ctx_hbm if 1 <= j < HDR else table_hbm

    def start_gather(j):
        b = j % NB
        pltpu.async_copy(src(j).at[tjidx_v.at[pl.ds(j * cpw, cpw)]],
                         bufs[b], gsems[b])

    def wait_gather(j):
        b = j % NB
        pltpu.make_async_copy(src(j).at[tjidx_v.at[pl.ds(j * cpw, cpw)]],
                              bufs[b], gsems[b]).wait()

    def start_write(j):
        b = j % NB
        pltpu.async_copy(bufs[b], emb_hbm.at[j, pl.ds(base, cpw)], wsems[b])

    def wait_write(j):
        b = j % NB
        pltpu.make_async_copy(bufs[b], emb_hbm.at[j, pl.ds(base, cpw)],
                              wsems[b]).wait()

    for j in range(NB):
        start_gather(j)

    for g in range(cpw // L):
        def eof_body(j, mb, g=g):
            m, best = mb
            v = tok_v[pl.ds(j * cpw + g * L, L)]
            gt = v > m
            best = jnp.where(gt, jnp.full((L,), j, jnp.int32), best)
            m = jnp.maximum(m, v)
            return m, best

        m0 = jnp.full((L,), -1, jnp.int32)
        b0 = jnp.zeros((L,), jnp.int32)
        _, best = lax.fori_loop(0, CTX_LEN, eof_body, (m0, b0))
        eof_v[pl.ds(g * L, L)] = best

    pltpu.sync_copy(eof_v, eof_hbm.at[pl.ds(base, cpw)])

    for j in range(CTX_LEN):
        if j > 0 and j - 1 + NB < CTX_LEN:
            wait_write(j - 1)
            start_gather(j - 1 + NB)
        wait_gather(j)
        start_write(j)
    for j in range(CTX_LEN - NB, CTX_LEN):
        wait_write(j)


def kernel(tokenized_text_prototypes, token_embedding_table, ctx_vectors):
    tokens = tokenized_text_prototypes.astype(jnp.int32)

    info = plsc.get_sparse_core_info()
    nc, ns = info.num_cores, info.num_subcores
    nw = nc * ns
    cpw = ((N_CLS + nw - 1) // nw + 7) // 8 * 8

    base = jnp.minimum(jnp.arange(nw) * cpw, N_CLS - cpw)
    cls = base[:, None] + jnp.arange(cpw)[None, :]

    blk = tokens[cls]
    pref = jnp.broadcast_to(tokens[0, 0], (nw, 1, cpw))
    ctxi = jnp.broadcast_to(jnp.arange(M, dtype=jnp.int32)[None, :, None],
                            (nw, M, cpw))
    sufi = blk[:, :, HDR:].transpose(0, 2, 1)
    tjidx = jnp.concatenate([pref, ctxi, sufi], axis=1)

    tokt = jnp.pad(blk, ((0, 0), (0, 0), (0, TOKP - CTX_LEN)),
                   constant_values=-1).transpose(0, 2, 1)

    mesh = plsc.VectorSubcoreMesh(core_axis_name="c", subcore_axis_name="s",
                                  num_cores=nc, num_subcores=ns)
    fn = pl.kernel(
        functools.partial(_body, nc=nc, cpw=cpw),
        out_type=(
            jax.ShapeDtypeStruct((CTX_LEN, N_CLS, D), jnp.float32),
            jax.ShapeDtypeStruct((N_CLS,), jnp.int32),
        ),
        mesh=mesh,
        scratch_types=[
            [pltpu.VMEM((cpw, D), jnp.float32) for _ in range(NB)],
            pltpu.VMEM((CTX_LEN * cpw,), jnp.int32),
            pltpu.VMEM((TOKP * cpw,), jnp.int32),
            pltpu.VMEM((cpw,), jnp.int32),
            [pltpu.SemaphoreType.DMA for _ in range(NB)],
            [pltpu.SemaphoreType.DMA for _ in range(NB)],
        ],
        compiler_params=pltpu.CompilerParams(use_tc_tiling_on_sc=True),
    )
    emb77, eofs = fn(tjidx.reshape(-1), tokt.reshape(-1),
                     token_embedding_table, ctx_vectors)
    return jnp.transpose(emb77, (1, 0, 2)), eofs


# speedup vs baseline: 3.1443x; 1.0325x over previous
import functools

import jax
import jax.numpy as jnp
from jax import lax
from jax.experimental import pallas as pl
from jax.experimental.pallas import tpu as pltpu
from jax.experimental.pallas import tpu_sc as plsc

VOCAB = 49408
D = 512
N_CLS = 1000
CTX_LEN = 77
M = 16
HDR = M + 1
TOKP = 80
L = 16


NB = 6


def _body(tjidx_hbm, tokt_hbm, table_hbm, ctx_hbm,
          emb_hbm, eof_hbm,
          bufs, tjidx_v, tok_v, eof_v, gsems, wsems,
          *, nc, cpw):
    wid = lax.axis_index("s") * nc + lax.axis_index("c")
    base = jnp.minimum(wid * cpw, N_CLS - cpw)

    pltpu.sync_copy(tjidx_hbm.at[pl.ds(wid * (CTX_LEN * cpw), CTX_LEN * cpw)],
                    tjidx_v)
    pltpu.sync_copy(tokt_hbm.at[pl.ds(wid * (TOKP * cpw), TOKP * cpw)], tok_v)

    def src(j):
        return ctx_hbm if 1 <= j < HDR else table_hbm

    def start_gather(j):
        b = j % NB
        pltpu.async_copy(src(j).at[tjidx_v.at[pl.ds(j * cpw, cpw)]],
                         bufs[b], gsems[b])

    def wait_gather(j):
        b = j % NB
        pltpu.make_async_copy(src(j).at[tjidx_v.at[pl.ds(j * cpw, cpw)]],
                              bufs[b], gsems[b]).wait()

    def start_write(j):
        b = j % NB
        pltpu.async_copy(bufs[b], emb_hbm.at[j, pl.ds(base, cpw)], wsems[b])

    def wait_write(j):
        b = j % NB
        pltpu.make_async_copy(bufs[b], emb_hbm.at[j, pl.ds(base, cpw)],
                              wsems[b]).wait()

    for j in range(NB):
        start_gather(j)

    for g in range(cpw // L):
        def eof_body(j, mb, g=g):
            m, best = mb
            v = tok_v[pl.ds(j * cpw + g * L, L)]
            gt = v > m
            best = jnp.where(gt, jnp.full((L,), j, jnp.int32), best)
            m = jnp.maximum(m, v)
            return m, best

        m0 = jnp.full((L,), -1, jnp.int32)
        b0 = jnp.zeros((L,), jnp.int32)
        _, best = lax.fori_loop(0, CTX_LEN, eof_body, (m0, b0))
        eof_v[pl.ds(g * L, L)] = best

    pltpu.sync_copy(eof_v, eof_hbm.at[pl.ds(base, cpw)])

    for j in range(CTX_LEN):
        if j > 0 and j - 1 + NB < CTX_LEN:
            wait_write(j - 1)
            start_gather(j - 1 + NB)
        wait_gather(j)
        start_write(j)
    for j in range(CTX_LEN - NB, CTX_LEN):
        wait_write(j)


def kernel(tokenized_text_prototypes, token_embedding_table, ctx_vectors):
    tokens = tokenized_text_prototypes.astype(jnp.int32)

    info = plsc.get_sparse_core_info()
    nc, ns = info.num_cores, info.num_subcores
    nw = nc * ns
    cpw = ((N_CLS + nw - 1) // nw + 7) // 8 * 8

    base = jnp.minimum(jnp.arange(nw) * cpw, N_CLS - cpw)
    cls = base[:, None] + jnp.arange(cpw)[None, :]

    blk = tokens[cls]
    pref = jnp.broadcast_to(tokens[0, 0], (nw, 1, cpw))
    ctxi = jnp.broadcast_to(jnp.arange(M, dtype=jnp.int32)[None, :, None],
                            (nw, M, cpw))
    sufi = blk[:, :, HDR:].transpose(0, 2, 1)
    tjidx = jnp.concatenate([pref, ctxi, sufi], axis=1)

    tokt = jnp.pad(blk, ((0, 0), (0, 0), (0, TOKP - CTX_LEN)),
                   constant_values=-1).transpose(0, 2, 1)

    mesh = plsc.VectorSubcoreMesh(core_axis_name="c", subcore_axis_name="s",
                                  num_cores=nc, num_subcores=ns)
    fn = pl.kernel(
        functools.partial(_body, nc=nc, cpw=cpw),
        out_type=(
            jax.ShapeDtypeStruct((CTX_LEN, N_CLS, D), jnp.float32),
            jax.ShapeDtypeStruct((N_CLS,), jnp.int32),
        ),
        mesh=mesh,
        scratch_types=[
            [pltpu.VMEM((cpw, D), jnp.float32) for _ in range(NB)],
            pltpu.VMEM((CTX_LEN * cpw,), jnp.int32),
            pltpu.VMEM((TOKP * cpw,), jnp.int32),
            pltpu.VMEM((cpw,), jnp.int32),
            [pltpu.SemaphoreType.DMA for _ in range(NB)],
            [pltpu.SemaphoreType.DMA for _ in range(NB)],
        ],
        compiler_params=pltpu.CompilerParams(use_tc_tiling_on_sc=True),
    )
    emb77, eofs = fn(tjidx.reshape(-1), tokt.reshape(-1),
                     token_embedding_table, ctx_vectors)
    return jnp.transpose(emb77, (1, 0, 2)), eofs


# speedup vs baseline: 3.1599x; 1.0050x over previous
import functools

import jax
import jax.numpy as jnp
from jax import lax
from jax.experimental import pallas as pl
from jax.experimental.pallas import tpu as pltpu
from jax.experimental.pallas import tpu_sc as plsc

VOCAB = 49408
D = 512
N_CLS = 1000
CTX_LEN = 77
M = 16
HDR = M + 1
TOKP = 80
L = 16


NB = 7


def _body(tjidx_hbm, tokt_hbm, table_hbm, ctx_hbm,
          emb_hbm, eof_hbm,
          bufs, tjidx_v, tok_v, eof_v, gsems, wsems,
          *, nc, cpw):
    wid = lax.axis_index("s") * nc + lax.axis_index("c")
    base = jnp.minimum(wid * cpw, N_CLS - cpw)

    pltpu.sync_copy(tjidx_hbm.at[pl.ds(wid * (CTX_LEN * cpw), CTX_LEN * cpw)],
                    tjidx_v)
    pltpu.sync_copy(tokt_hbm.at[pl.ds(wid * (TOKP * cpw), TOKP * cpw)], tok_v)

    def src(j):
        return ctx_hbm if 1 <= j < HDR else table_hbm

    def start_gather(j):
        b = j % NB
        pltpu.async_copy(src(j).at[tjidx_v.at[pl.ds(j * cpw, cpw)]],
                         bufs[b], gsems[b])

    def wait_gather(j):
        b = j % NB
        pltpu.make_async_copy(src(j).at[tjidx_v.at[pl.ds(j * cpw, cpw)]],
                              bufs[b], gsems[b]).wait()

    def start_write(j):
        b = j % NB
        pltpu.async_copy(bufs[b], emb_hbm.at[j, pl.ds(base, cpw)], wsems[b])

    def wait_write(j):
        b = j % NB
        pltpu.make_async_copy(bufs[b], emb_hbm.at[j, pl.ds(base, cpw)],
                              wsems[b]).wait()

    for j in range(NB):
        start_gather(j)

    for g in range(cpw // L):
        def eof_body(j, mb, g=g):
            m, best = mb
            v = tok_v[pl.ds(j * cpw + g * L, L)]
            gt = v > m
            best = jnp.where(gt, jnp.full((L,), j, jnp.int32), best)
            m = jnp.maximum(m, v)
            return m, best

        m0 = jnp.full((L,), -1, jnp.int32)
        b0 = jnp.zeros((L,), jnp.int32)
        _, best = lax.fori_loop(0, CTX_LEN, eof_body, (m0, b0))
        eof_v[pl.ds(g * L, L)] = best

    pltpu.sync_copy(eof_v, eof_hbm.at[pl.ds(base, cpw)])

    for j in range(CTX_LEN):
        if j > 0 and j - 1 + NB < CTX_LEN:
            wait_write(j - 1)
            start_gather(j - 1 + NB)
        wait_gather(j)
        start_write(j)
    for j in range(CTX_LEN - NB, CTX_LEN):
        wait_write(j)


def kernel(tokenized_text_prototypes, token_embedding_table, ctx_vectors):
    tokens = tokenized_text_prototypes.astype(jnp.int32)

    info = plsc.get_sparse_core_info()
    nc, ns = info.num_cores, info.num_subcores
    nw = nc * ns
    cpw = ((N_CLS + nw - 1) // nw + 7) // 8 * 8

    base = jnp.minimum(jnp.arange(nw) * cpw, N_CLS - cpw)
    cls = base[:, None] + jnp.arange(cpw)[None, :]

    blk = tokens[cls]
    pref = jnp.broadcast_to(tokens[0, 0], (nw, 1, cpw))
    ctxi = jnp.broadcast_to(jnp.arange(M, dtype=jnp.int32)[None, :, None],
                            (nw, M, cpw))
    sufi = blk[:, :, HDR:].transpose(0, 2, 1)
    tjidx = jnp.concatenate([pref, ctxi, sufi], axis=1)

    tokt = jnp.pad(blk, ((0, 0), (0, 0), (0, TOKP - CTX_LEN)),
                   constant_values=-1).transpose(0, 2, 1)

    mesh = plsc.VectorSubcoreMesh(core_axis_name="c", subcore_axis_name="s",
                                  num_cores=nc, num_subcores=ns)
    fn = pl.kernel(
        functools.partial(_body, nc=nc, cpw=cpw),
        out_type=(
            jax.ShapeDtypeStruct((CTX_LEN, N_CLS, D), jnp.float32),
            jax.ShapeDtypeStruct((N_CLS,), jnp.int32),
        ),
        mesh=mesh,
        scratch_types=[
            [pltpu.VMEM((cpw, D), jnp.float32) for _ in range(NB)],
            pltpu.VMEM((CTX_LEN * cpw,), jnp.int32),
            pltpu.VMEM((TOKP * cpw,), jnp.int32),
            pltpu.VMEM((cpw,), jnp.int32),
            [pltpu.SemaphoreType.DMA for _ in range(NB)],
            [pltpu.SemaphoreType.DMA for _ in range(NB)],
        ],
        compiler_params=pltpu.CompilerParams(use_tc_tiling_on_sc=True),
    )
    emb77, eofs = fn(tjidx.reshape(-1), tokt.reshape(-1),
                     token_embedding_table, ctx_vectors)
    return jnp.transpose(emb77, (1, 0, 2)), eofs
